# Initial kernel scaffold; baseline (speedup 1.0000x reference)
import jax
import jax.numpy as jnp
from jax.experimental import pallas as pl


def kernel(input):
    raise NotImplementedError("write your pallas kernel here")


# speedup vs baseline: 16.0866x; 16.0866x over previous
import functools

import jax
import jax.numpy as jnp
from jax import lax
from jax.experimental import pallas as pl
from jax.experimental.pallas import tpu as pltpu
from jax.experimental.pallas import tpu_sc as plsc

ROWS = 32768
COLS = 2048
N = ROWS * COLS

NUM_CORES = 2
NUM_SUBCORES = 16
NUM_WORKERS = NUM_CORES * NUM_SUBCORES
PER_WORKER = N // NUM_WORKERS

CHUNK = 16384
N_CHUNKS = PER_WORKER // CHUNK

_mesh = plsc.VectorSubcoreMesh(core_axis_name="c", subcore_axis_name="s")


@functools.partial(
    pl.kernel,
    out_type=jax.ShapeDtypeStruct((N,), jnp.float32),
    mesh=_mesh,
    scratch_types=[
        pltpu.VMEM((CHUNK,), jnp.float32),
        pltpu.VMEM((CHUNK,), jnp.float32),
    ],
    compiler_params=pltpu.CompilerParams(needs_layout_passes=False),
)
def _group_sort_sc(x_hbm, o_hbm, b_in, b_out):
    wid = lax.axis_index("s") * NUM_CORES + lax.axis_index("c")
    w_base = wid * PER_WORKER
    lane = lax.iota(jnp.int32, 16)
    even0 = lane * 2

    def chunk_body(i, carry):
        base = w_base + i * CHUNK
        pltpu.sync_copy(x_hbm.at[pl.ds(base, CHUNK)], b_in)

        def span_body(j, carry2):
            ie = even0 + j * 32
            io = ie + 1
            a = plsc.load_gather(b_in, [ie])
            b = plsc.load_gather(b_in, [io])
            plsc.store_scatter(b_out, [ie], jnp.minimum(a, b))
            plsc.store_scatter(b_out, [io], jnp.maximum(a, b))
            return carry2

        lax.fori_loop(0, CHUNK // 32, span_body, 0)
        pltpu.sync_copy(b_out, o_hbm.at[pl.ds(base, CHUNK)])
        return carry

    lax.fori_loop(0, N_CHUNKS, chunk_body, 0)


def kernel(input):
    flat = input.reshape(-1)
    out = _group_sort_sc(flat)
    return out.reshape(input.shape)


# speedup vs baseline: 27.1903x; 1.6902x over previous
import functools

import jax
import jax.numpy as jnp
from jax import lax
from jax.experimental import pallas as pl
from jax.experimental.pallas import tpu as pltpu
from jax.experimental.pallas import tpu_sc as plsc

ROWS = 32768
COLS = 2048
N = ROWS * COLS

NUM_CORES = 2
NUM_SUBCORES = 16
NUM_WORKERS = NUM_CORES * NUM_SUBCORES
PER_WORKER = N // NUM_WORKERS

CHUNK = 16384
N_CHUNKS = PER_WORKER // CHUNK


_mesh = plsc.VectorSubcoreMesh(core_axis_name="c", subcore_axis_name="s")


@functools.partial(
    pl.kernel,
    out_type=jax.ShapeDtypeStruct((N,), jnp.float32),
    mesh=_mesh,
    scratch_types=[
        pltpu.VMEM((CHUNK,), jnp.float32),
        pltpu.VMEM((CHUNK,), jnp.float32),
        pltpu.VMEM((CHUNK,), jnp.float32),
        pltpu.VMEM((CHUNK,), jnp.float32),
        pltpu.SemaphoreType.DMA,
        pltpu.SemaphoreType.DMA,
        pltpu.SemaphoreType.DMA,
        pltpu.SemaphoreType.DMA,
    ],
    compiler_params=pltpu.CompilerParams(needs_layout_passes=False),
)
def _group_sort_sc(x_hbm, o_hbm, bi0, bi1, bo0, bo1, si0, si1, so0, so1):
    wid = lax.axis_index("s") * NUM_CORES + lax.axis_index("c")
    w_base = wid * PER_WORKER
    lane = lax.iota(jnp.int32, 16)
    even0 = lane * 2

    b_in = (bi0, bi1)
    b_out = (bo0, bo1)
    s_in = (si0, si1)
    s_out = (so0, so1)

    def start_in(g, p):
        pltpu.async_copy(
            x_hbm.at[pl.ds(w_base + g * CHUNK, CHUNK)], b_in[p], s_in[p])

    def wait_in(p):
        pltpu.make_async_copy(
            x_hbm.at[pl.ds(w_base, CHUNK)], b_in[p], s_in[p]).wait()

    def start_out(g, p):
        pltpu.async_copy(
            b_out[p], o_hbm.at[pl.ds(w_base + g * CHUNK, CHUNK)], s_out[p])

    def wait_out(p):
        pltpu.make_async_copy(
            b_out[p], o_hbm.at[pl.ds(w_base, CHUNK)], s_out[p]).wait()

    def compute(p):
        src = b_in[p]
        dst = b_out[p]

        @plsc.parallel_loop(0, CHUNK // 32, unroll=8)
        def _(j):
            ie = even0 + j * 32
            io = ie + 1
            a = plsc.load_gather(src, [ie])
            b = plsc.load_gather(src, [io])
            plsc.store_scatter(dst, [ie], jnp.minimum(a, b))
            plsc.store_scatter(dst, [io], jnp.maximum(a, b))

    start_in(0, 0)

    def pair_body(gp, carry):
        for p in range(2):
            g = gp * 2 + p

            @pl.when(g + 1 < N_CHUNKS)
            def _():
                start_in(g + 1, 1 - p)

            wait_in(p)

            @pl.when(gp > 0)
            def _():
                wait_out(p)

            compute(p)
            start_out(g, p)
        return carry

    lax.fori_loop(0, N_CHUNKS // 2, pair_body, 0)
    wait_out(0)
    wait_out(1)


def kernel(input):
    flat = input.reshape(-1)
    out = _group_sort_sc(flat)
    return out.reshape(input.shape)


# speedup vs baseline: 84.1233x; 3.0939x over previous
import functools

import jax
import jax.numpy as jnp
from jax import lax
from jax.experimental import pallas as pl
from jax.experimental.pallas import tpu as pltpu
from jax.experimental.pallas import tpu_sc as plsc

ROWS = 32768
COLS = 2048

NUM_CORES = 2
NUM_SUBCORES = 16
NUM_WORKERS = NUM_CORES * NUM_SUBCORES
ROWS_PER_WORKER = ROWS // NUM_WORKERS

CHUNK_ROWS = 8
N_CHUNKS = ROWS_PER_WORKER // CHUNK_ROWS
SPANS_PER_ROW = COLS // 32


_mesh = plsc.VectorSubcoreMesh(core_axis_name="c", subcore_axis_name="s")


@functools.partial(
    pl.kernel,
    out_type=jax.ShapeDtypeStruct((ROWS, COLS), jnp.float32),
    mesh=_mesh,
    scratch_types=[
        pltpu.VMEM((CHUNK_ROWS, COLS), jnp.float32),
        pltpu.VMEM((CHUNK_ROWS, COLS), jnp.float32),
        pltpu.VMEM((CHUNK_ROWS, COLS), jnp.float32),
        pltpu.VMEM((CHUNK_ROWS, COLS), jnp.float32),
        pltpu.SemaphoreType.DMA,
        pltpu.SemaphoreType.DMA,
        pltpu.SemaphoreType.DMA,
        pltpu.SemaphoreType.DMA,
    ],
    compiler_params=pltpu.CompilerParams(needs_layout_passes=False),
)
def _group_sort_sc(x_hbm, o_hbm, bi0, bi1, bo0, bo1, si0, si1, so0, so1):
    wid = lax.axis_index("s") * NUM_CORES + lax.axis_index("c")
    w_row = wid * ROWS_PER_WORKER
    lane = lax.iota(jnp.int32, 16)
    even0 = lane * 2

    b_in = (bi0, bi1)
    b_out = (bo0, bo1)
    s_in = (si0, si1)
    s_out = (so0, so1)

    def start_in(g, p):
        pltpu.async_copy(
            x_hbm.at[pl.ds(w_row + g * CHUNK_ROWS, CHUNK_ROWS), :],
            b_in[p], s_in[p])

    def wait_in(p):
        pltpu.make_async_copy(
            x_hbm.at[pl.ds(w_row, CHUNK_ROWS), :], b_in[p], s_in[p]).wait()

    def start_out(g, p):
        pltpu.async_copy(
            b_out[p],
            o_hbm.at[pl.ds(w_row + g * CHUNK_ROWS, CHUNK_ROWS), :], s_out[p])

    def wait_out(p):
        pltpu.make_async_copy(
            b_out[p], o_hbm.at[pl.ds(w_row, CHUNK_ROWS), :], s_out[p]).wait()

    def compute(p):
        src = b_in[p]
        dst = b_out[p]
        for r in range(CHUNK_ROWS):
            row_vec = jnp.full((16,), r, jnp.int32)

            @plsc.parallel_loop(0, SPANS_PER_ROW, unroll=8)
            def _(j):
                ce = even0 + j * 32
                co = ce + 1
                a = plsc.load_gather(src, [row_vec, ce])
                b = plsc.load_gather(src, [row_vec, co])
                plsc.store_scatter(dst, [row_vec, ce], jnp.minimum(a, b))
                plsc.store_scatter(dst, [row_vec, co], jnp.maximum(a, b))

    start_in(0, 0)

    def pair_body(gp, carry):
        for p in range(2):
            g = gp * 2 + p

            @pl.when(g + 1 < N_CHUNKS)
            def _():
                start_in(g + 1, 1 - p)

            wait_in(p)

            @pl.when(gp > 0)
            def _():
                wait_out(p)

            compute(p)
            start_out(g, p)
        return carry

    lax.fori_loop(0, N_CHUNKS // 2, pair_body, 0)
    wait_out(0)
    wait_out(1)


def kernel(input):
    return _group_sort_sc(input)


# speedup vs baseline: 86.1398x; 1.0240x over previous
import functools

import jax
import jax.numpy as jnp
from jax import lax
from jax.experimental import pallas as pl
from jax.experimental.pallas import tpu as pltpu
from jax.experimental.pallas import tpu_sc as plsc

ROWS = 32768
COLS = 2048

NUM_CORES = 2
NUM_SUBCORES = 16
NUM_WORKERS = NUM_CORES * NUM_SUBCORES
ROWS_PER_WORKER = ROWS // NUM_WORKERS

CHUNK_ROWS = 16
N_CHUNKS = ROWS_PER_WORKER // CHUNK_ROWS
SPANS_PER_ROW = COLS // 32
SPANS_PER_CHUNK = CHUNK_ROWS * SPANS_PER_ROW


_mesh = plsc.VectorSubcoreMesh(core_axis_name="c", subcore_axis_name="s")


@functools.partial(
    pl.kernel,
    out_type=jax.ShapeDtypeStruct((ROWS, COLS), jnp.float32),
    mesh=_mesh,
    scratch_types=[
        pltpu.VMEM((CHUNK_ROWS, COLS), jnp.float32),
        pltpu.VMEM((CHUNK_ROWS, COLS), jnp.float32),
        pltpu.VMEM((CHUNK_ROWS, COLS), jnp.float32),
        pltpu.SemaphoreType.DMA,
        pltpu.SemaphoreType.DMA,
        pltpu.SemaphoreType.DMA,
        pltpu.SemaphoreType.DMA,
        pltpu.SemaphoreType.DMA,
        pltpu.SemaphoreType.DMA,
    ],
    compiler_params=pltpu.CompilerParams(needs_layout_passes=False),
)
def _group_sort_sc(x_hbm, o_hbm, b0, b1, b2, si0, si1, si2, so0, so1, so2):
    wid = lax.axis_index("s") * NUM_CORES + lax.axis_index("c")
    w_row = wid * ROWS_PER_WORKER
    lane = lax.iota(jnp.int32, 16)
    even0 = lane * 2

    buf = (b0, b1, b2)
    s_in = (si0, si1, si2)
    s_out = (so0, so1, so2)

    def start_in(g, p):
        pltpu.async_copy(
            x_hbm.at[pl.ds(w_row + g * CHUNK_ROWS, CHUNK_ROWS), :],
            buf[p], s_in[p])

    def wait_in(p):
        pltpu.make_async_copy(
            x_hbm.at[pl.ds(w_row, CHUNK_ROWS), :], buf[p], s_in[p]).wait()

    def start_out(g, p):
        pltpu.async_copy(
            buf[p],
            o_hbm.at[pl.ds(w_row + g * CHUNK_ROWS, CHUNK_ROWS), :], s_out[p])

    def wait_out(p):
        pltpu.make_async_copy(
            buf[p], o_hbm.at[pl.ds(w_row, CHUNK_ROWS), :], s_out[p]).wait()

    def compute(p):
        b = buf[p]

        @plsc.parallel_loop(0, SPANS_PER_CHUNK, unroll=8)
        def _(j):
            r = j >> 6
            jc = j & (SPANS_PER_ROW - 1)
            row_vec = jnp.full((16,), 0, jnp.int32) + r
            ce = even0 + jc * 32
            co = ce + 1
            va = plsc.load_gather(b, [row_vec, ce])
            vb = plsc.load_gather(b, [row_vec, co])
            plsc.store_scatter(b, [row_vec, ce], jnp.minimum(va, vb))
            plsc.store_scatter(b, [row_vec, co], jnp.maximum(va, vb))

    start_in(0, 0)
    start_in(1, 1)

    def triple_body(gp, carry):
        for p in range(3):
            g = gp * 3 + p
            wait_in(p)
            compute(p)
            start_out(g, p)

            @pl.when(jnp.logical_and(g >= 1, g + 2 < N_CHUNKS))
            def _():
                wait_out((p + 2) % 3)

            @pl.when(g + 2 < N_CHUNKS)
            def _():
                start_in(g + 2, (p + 2) % 3)
        return carry

    lax.fori_loop(0, N_CHUNKS // 3, triple_body, 0)

    g_tail = N_CHUNKS - 1
    p_tail = g_tail % 3
    wait_in(p_tail)
    compute(p_tail)
    start_out(g_tail, p_tail)

    wait_out((g_tail + 1) % 3)
    wait_out((g_tail + 2) % 3)
    wait_out(p_tail)


def kernel(input):
    return _group_sort_sc(input)


# speedup vs baseline: 86.5446x; 1.0047x over previous
import functools

import jax
import jax.numpy as jnp
from jax import lax
from jax.experimental import pallas as pl
from jax.experimental.pallas import tpu as pltpu
from jax.experimental.pallas import tpu_sc as plsc

ROWS = 32768
COLS = 2048

NUM_CORES = 2
NUM_SUBCORES = 16
NUM_WORKERS = NUM_CORES * NUM_SUBCORES
ROWS_PER_WORKER = ROWS // NUM_WORKERS

CHUNK_ROWS = 8
N_CHUNKS = ROWS_PER_WORKER // CHUNK_ROWS
SPANS_PER_ROW = COLS // 32
SPANS_PER_CHUNK = CHUNK_ROWS * SPANS_PER_ROW

NBUF = 4


_mesh = plsc.VectorSubcoreMesh(core_axis_name="c", subcore_axis_name="s")


@functools.partial(
    pl.kernel,
    out_type=jax.ShapeDtypeStruct((ROWS, COLS), jnp.float32),
    mesh=_mesh,
    scratch_types=(
        [pltpu.VMEM((CHUNK_ROWS, COLS), jnp.float32)] * NBUF
        + [pltpu.SemaphoreType.DMA] * (2 * NBUF)
    ),
    compiler_params=pltpu.CompilerParams(needs_layout_passes=False),
)
def _group_sort_sc(x_hbm, o_hbm, *refs):
    buf = refs[:NBUF]
    s_in = refs[NBUF:2 * NBUF]
    s_out = refs[2 * NBUF:]

    wid = lax.axis_index("s") * NUM_CORES + lax.axis_index("c")
    w_row = wid * ROWS_PER_WORKER
    lane = lax.iota(jnp.int32, 16)
    even0 = lane * 2

    def start_in(g, p):
        pltpu.async_copy(
            x_hbm.at[pl.ds(w_row + g * CHUNK_ROWS, CHUNK_ROWS), :],
            buf[p], s_in[p])

    def wait_in(p):
        pltpu.make_async_copy(
            x_hbm.at[pl.ds(w_row, CHUNK_ROWS), :], buf[p], s_in[p]).wait()

    def start_out(g, p):
        pltpu.async_copy(
            buf[p],
            o_hbm.at[pl.ds(w_row + g * CHUNK_ROWS, CHUNK_ROWS), :], s_out[p])

    def wait_out(p):
        pltpu.make_async_copy(
            buf[p], o_hbm.at[pl.ds(w_row, CHUNK_ROWS), :], s_out[p]).wait()

    def compute(p):
        b = buf[p]

        @plsc.parallel_loop(0, SPANS_PER_CHUNK, unroll=8)
        def _(j):
            r = j >> 6
            jc = j & (SPANS_PER_ROW - 1)
            row_vec = jnp.full((16,), 0, jnp.int32) + r
            ce = even0 + jc * 32
            co = ce + 1
            va = plsc.load_gather(b, [row_vec, ce])
            vb = plsc.load_gather(b, [row_vec, co])
            plsc.store_scatter(b, [row_vec, ce], jnp.minimum(va, vb))
            plsc.store_scatter(b, [row_vec, co], jnp.maximum(va, vb))

    for p0 in range(NBUF - 1):
        start_in(p0, p0)

    def quad_body(gq, carry):
        for p in range(NBUF):
            g = gq * NBUF + p
            wait_in(p)
            compute(p)
            start_out(g, p)

            @pl.when(jnp.logical_and(g >= 1, g + 3 < N_CHUNKS))
            def _():
                wait_out((p + 3) % NBUF)

            @pl.when(g + 3 < N_CHUNKS)
            def _():
                start_in(g + 3, (p + 3) % NBUF)
        return carry

    lax.fori_loop(0, N_CHUNKS // NBUF, quad_body, 0)

    for p0 in range(NBUF):
        wait_out(p0)


def kernel(input):
    return _group_sort_sc(input)
